# Initial kernel scaffold; baseline (speedup 1.0000x reference)
import jax
import jax.numpy as jnp
from jax.experimental import pallas as pl


def kernel(x, edge_index, batch, W00, b00, g00, bt00, W01, b01, g01, bt01, W10, b10, g10, bt10, W11, b11, g11, bt11, fusion_W, fusion_b):
    raise NotImplementedError("write your pallas kernel here")


# speedup vs baseline: 12.2756x; 12.2756x over previous
import functools

import jax
import jax.numpy as jnp
from jax import lax
from jax.experimental import pallas as pl
from jax.experimental.pallas import tpu as pltpu
from jax.experimental.pallas import tpu_sc as plsc

EPS = 1e-5
NC = 2
NS = 16
NW = NC * NS
CH = 128


@functools.partial(jax.jit, static_argnames=("n_pad", "k_per_w"))
def _sc_scatter(g, src2d, dst2d, zblk, *, n_pad, k_per_w):
  d = 128
  stripe = n_pad // NS
  nz = stripe // CH
  mesh = plsc.VectorSubcoreMesh(core_axis_name="c", subcore_axis_name="s")

  @functools.partial(
      pl.kernel,
      out_type=jax.ShapeDtypeStruct((NC, n_pad, d), jnp.float32),
      mesh=mesh,
      scratch_types=[
          pltpu.VMEM_SHARED((n_pad, d), jnp.float32),
          pltpu.VMEM((CH,), jnp.int32),
          pltpu.VMEM((CH,), jnp.int32),
          pltpu.VMEM((CH, d), jnp.float32),
          pltpu.VMEM((CH, d), jnp.float32),
          pltpu.SemaphoreType.DMA,
      ],
  )
  def k(g_hbm, src_hbm, dst_hbm, z_hbm, out_hbm,
        acc_sh, src_v, dst_v, rows_v, zbuf_v, sem):
    c = lax.axis_index("c")
    s = lax.axis_index("s")
    wid = s * NC + c

    pltpu.sync_copy(z_hbm, zbuf_v)
    for i in range(nz):
      pltpu.sync_copy(zbuf_v, acc_sh.at[pl.ds(s * stripe + i * CH, CH)])
    plsc.subcore_barrier()

    base = wid * k_per_w

    def step(t, carry):
      pltpu.sync_copy(src_hbm.at[base + t], src_v)
      pltpu.sync_copy(dst_hbm.at[base + t], dst_v)
      pltpu.async_copy(g_hbm.at[src_v], rows_v, sem).wait()
      pltpu.sync_copy(rows_v, acc_sh.at[dst_v], add=True)
      return carry

    lax.fori_loop(0, k_per_w, step, 0)
    plsc.subcore_barrier()

    for i in range(nz):
      r0 = s * stripe + i * CH
      pltpu.sync_copy(acc_sh.at[pl.ds(r0, CH)], rows_v)
      pltpu.sync_copy(rows_v, out_hbm.at[c, pl.ds(r0, CH)])

  return k(g, src2d, dst2d, zblk)


@functools.partial(jax.jit, static_argnames=("n_pad", "k_per_w"))
def _sc_degree(dst2d, ones_blk, zblk16, *, n_pad, k_per_w):
  d = 128
  stripe = n_pad // NS
  nz = stripe // CH
  mesh = plsc.VectorSubcoreMesh(core_axis_name="c", subcore_axis_name="s")

  @functools.partial(
      pl.kernel,
      out_type=jax.ShapeDtypeStruct((NC, n_pad, d), jnp.float32),
      mesh=mesh,
      scratch_types=[
          pltpu.VMEM_SHARED((n_pad, d), jnp.float32),
          pltpu.VMEM((CH,), jnp.int32),
          pltpu.VMEM((CH, d), jnp.float32),
          pltpu.VMEM((CH, d), jnp.float32),
      ],
  )
  def k(dst_hbm, ones_hbm, z_hbm, out_hbm, acc_sh, dst_v, ones_v, zbuf_v):
    c = lax.axis_index("c")
    s = lax.axis_index("s")
    wid = s * NC + c

    pltpu.sync_copy(z_hbm, zbuf_v)
    for i in range(nz):
      pltpu.sync_copy(zbuf_v, acc_sh.at[pl.ds(s * stripe + i * CH, CH)])
    pltpu.sync_copy(ones_hbm, ones_v)
    plsc.subcore_barrier()

    base = wid * k_per_w

    def step(t, carry):
      pltpu.sync_copy(dst_hbm.at[base + t], dst_v)
      pltpu.sync_copy(ones_v, acc_sh.at[dst_v], add=True)
      return carry

    lax.fori_loop(0, k_per_w, step, 0)
    plsc.subcore_barrier()

    for i in range(nz):
      r0 = s * stripe + i * CH
      pltpu.sync_copy(acc_sh.at[pl.ds(r0, CH)], zbuf_v)
      pltpu.sync_copy(zbuf_v, out_hbm.at[c, pl.ds(r0, CH)])

  return k(dst2d, ones_blk, zblk16)


def _tc_first_body(degp_ref, x_ref, w_ref, dis_ref, g_ref):
  n = x_ref.shape[0]
  deg = degp_ref[0, :n, 0:1] + degp_ref[1, :n, 0:1] + 1.0
  dis = lax.rsqrt(deg)
  dis_ref[...] = dis
  g_ref[...] = dis * jnp.dot(x_ref[...], w_ref[...],
                             preferred_element_type=jnp.float32)


@jax.jit
def _tc_first(degp, x, w):
  n = x.shape[0]
  return pl.pallas_call(
      _tc_first_body,
      out_shape=(
          jax.ShapeDtypeStruct((n, 1), jnp.float32),
          jax.ShapeDtypeStruct((n, w.shape[1]), jnp.float32),
      ),
  )(degp, x, w)


def _bn_relu(p_ref, g_ref, dis_ref, b_ref, gm_ref, bt_ref):
  n = g_ref.shape[0]
  t = dis_ref[...] * (p_ref[0, :n] + p_ref[1, :n] + g_ref[...]) + b_ref[...]
  mu = jnp.mean(t, axis=0, keepdims=True)
  ctr = t - mu
  var = jnp.mean(ctr * ctr, axis=0, keepdims=True)
  return jnp.maximum(ctr * lax.rsqrt(var + EPS) * gm_ref[...] + bt_ref[...],
                     0.0)


def _pool(h, batch_ref):
  nb = batch_ref.shape[1]
  m = (batch_ref[...] == lax.broadcasted_iota(jnp.int32, (128, nb), 0))
  m = m.astype(jnp.float32)
  cnt = jnp.sum(m, axis=1, keepdims=True)
  return jnp.dot(m, h, preferred_element_type=jnp.float32) / jnp.maximum(
      cnt, 1.0)


def _tc_mid_body(p_ref, g_ref, dis_ref, b_ref, gm_ref, bt_ref, wn_ref,
                 gn_ref):
  h = _bn_relu(p_ref, g_ref, dis_ref, b_ref, gm_ref, bt_ref)
  gn_ref[...] = dis_ref[...] * jnp.dot(h, wn_ref[...],
                                       preferred_element_type=jnp.float32)


@jax.jit
def _tc_mid(p, g, dis, b, gm, bt, wn):
  return pl.pallas_call(
      _tc_mid_body,
      out_shape=jax.ShapeDtypeStruct(g.shape, jnp.float32),
  )(p, g, dis, b, gm, bt, wn)


def _tc_pool_mid_body(p_ref, g_ref, dis_ref, b_ref, gm_ref, bt_ref,
                      batch_ref, wn_ref, gn_ref, pool_ref):
  h = _bn_relu(p_ref, g_ref, dis_ref, b_ref, gm_ref, bt_ref)
  pool_ref[...] = _pool(h, batch_ref)
  gn_ref[...] = dis_ref[...] * jnp.dot(h, wn_ref[...],
                                       preferred_element_type=jnp.float32)


@jax.jit
def _tc_pool_mid(p, g, dis, b, gm, bt, batch2d, wn):
  return pl.pallas_call(
      _tc_pool_mid_body,
      out_shape=(
          jax.ShapeDtypeStruct(g.shape, jnp.float32),
          jax.ShapeDtypeStruct((128, g.shape[1]), jnp.float32),
      ),
  )(p, g, dis, b, gm, bt, batch2d, wn)


def _tc_final_body(p_ref, g_ref, dis_ref, b_ref, gm_ref, bt_ref, batch_ref,
                   pool1_ref, fw_ref, fb_ref, out_ref):
  h = _bn_relu(p_ref, g_ref, dis_ref, b_ref, gm_ref, bt_ref)
  pool2 = _pool(h, batch_ref)
  fused = jnp.concatenate([pool1_ref[...], pool2], axis=1)
  out_ref[...] = jnp.maximum(
      jnp.dot(fused, fw_ref[...], preferred_element_type=jnp.float32)
      + fb_ref[...], 0.0)


@jax.jit
def _tc_final(p, g, dis, b, gm, bt, batch2d, pool1, fw, fb):
  return pl.pallas_call(
      _tc_final_body,
      out_shape=jax.ShapeDtypeStruct((128, fw.shape[1]), jnp.float32),
  )(p, g, dis, b, gm, bt, batch2d, pool1, fw, fb)


def kernel(x, edge_index, batch,
           W00, b00, g00, bt00, W01, b01, g01, bt01,
           W10, b10, g10, bt10, W11, b11, g11, bt11,
           fusion_W, fusion_b):
  n, d = x.shape
  e = edge_index.shape[1]

  k_per_w = -(-e // (NW * CH))
  e_pad = NW * k_per_w * CH
  pad = e_pad - e
  n_pad = n + (NS * CH - n % (NS * CH)) % (NS * CH) if n % (NS * CH) else n
  if n_pad - n < 1:
    n_pad += NS * CH
  trash = n_pad - n

  pad_src = jnp.arange(pad, dtype=jnp.int32) % n
  pad_dst = n + jnp.arange(pad, dtype=jnp.int32) % trash
  src2d = jnp.concatenate([edge_index[0], pad_src]).reshape(e_pad // CH, CH)
  dst2d = jnp.concatenate(
      [edge_index[1], pad_dst.astype(jnp.int32)]).reshape(e_pad // CH, CH)

  zblk = jnp.zeros((CH, d), jnp.float32)
  ones_blk = jnp.ones((CH, d), jnp.float32)
  batch2d = batch.reshape(1, n)

  degp = _sc_degree(dst2d, ones_blk, zblk, n_pad=n_pad, k_per_w=k_per_w)

  dis, g = _tc_first(degp, x, W00)

  p = _sc_scatter(g, src2d, dst2d, zblk, n_pad=n_pad, k_per_w=k_per_w)
  g = _tc_mid(p, g, dis, b00.reshape(1, -1), g00.reshape(1, -1),
              bt00.reshape(1, -1), W01)

  p = _sc_scatter(g, src2d, dst2d, zblk, n_pad=n_pad, k_per_w=k_per_w)
  g, pool1 = _tc_pool_mid(p, g, dis, b01.reshape(1, -1),
                          g01.reshape(1, -1), bt01.reshape(1, -1),
                          batch2d, W10)

  p = _sc_scatter(g, src2d, dst2d, zblk, n_pad=n_pad, k_per_w=k_per_w)
  g = _tc_mid(p, g, dis, b10.reshape(1, -1), g10.reshape(1, -1),
              bt10.reshape(1, -1), W11)

  p = _sc_scatter(g, src2d, dst2d, zblk, n_pad=n_pad, k_per_w=k_per_w)
  out = _tc_final(p, g, dis, b11.reshape(1, -1),
                  g11.reshape(1, -1), bt11.reshape(1, -1), batch2d, pool1,
                  fusion_W, fusion_b.reshape(1, -1))
  return out
